# Initial kernel scaffold; baseline (speedup 1.0000x reference)
import jax
import jax.numpy as jnp
from jax.experimental import pallas as pl


def kernel(x, sorted_index, Wq, bq, Wk, bk, Wv, bv, Wo, bo):
    raise NotImplementedError("write your pallas kernel here")


# speedup vs baseline: 1.0461x; 1.0461x over previous
import jax
import jax.numpy as jnp
import numpy as np
from jax.experimental import pallas as pl

EMBED = 768
HEADS = 12
DH = 64
LQ = 1024
LK = 4096
BATCH = 4
SCALE = 1.0 / np.sqrt(DH)


def _attn_kernel(x_ref, wq_ref, wk_ref, wv_ref, wo_ref, bq_ref, cv_ref, o_ref):
    h = pl.program_id(1)
    xb = x_ref[0]
    q = jnp.dot(xb[:LQ], wq_ref[0], preferred_element_type=jnp.float32)
    q = q + bq_ref[pl.ds(h, 1), :]
    k = jnp.dot(xb, wk_ref[0], preferred_element_type=jnp.float32)
    v = jnp.dot(xb, wv_ref[0], preferred_element_type=jnp.float32)
    s = jnp.dot(q, k.T, preferred_element_type=jnp.float32) * SCALE
    m = jnp.max(s, axis=-1, keepdims=True)
    p = jnp.exp(s - m)
    denom = jnp.sum(p, axis=-1, keepdims=True)
    oh = jnp.dot(p, v, preferred_element_type=jnp.float32) / denom
    contrib = jnp.dot(oh, wo_ref[0], preferred_element_type=jnp.float32)

    @pl.when(h == 0)
    def _init():
        o_ref[...] = cv_ref[...] + contrib[None]

    @pl.when(h != 0)
    def _acc():
        o_ref[...] += contrib[None]


def kernel(x, sorted_index, Wq, bq, Wk, bk, Wv, bv, Wo, bo):
    del bk
    cv = (bv @ Wo + bo).reshape(1, EMBED)
    bq2 = bq.reshape(HEADS, DH)
    wq3 = Wq.reshape(EMBED, HEADS, DH).transpose(1, 0, 2)
    wk3 = Wk.reshape(EMBED, HEADS, DH).transpose(1, 0, 2)
    wv3 = Wv.reshape(EMBED, HEADS, DH).transpose(1, 0, 2)
    wo3 = Wo.reshape(HEADS, DH, EMBED)
    out = pl.pallas_call(
        _attn_kernel,
        grid=(BATCH, HEADS),
        in_specs=[
            pl.BlockSpec((1, LK, EMBED), lambda b, h: (b, 0, 0)),
            pl.BlockSpec((1, EMBED, DH), lambda b, h: (h, 0, 0)),
            pl.BlockSpec((1, EMBED, DH), lambda b, h: (h, 0, 0)),
            pl.BlockSpec((1, EMBED, DH), lambda b, h: (h, 0, 0)),
            pl.BlockSpec((1, DH, EMBED), lambda b, h: (h, 0, 0)),
            pl.BlockSpec((HEADS, DH), lambda b, h: (0, 0)),
            pl.BlockSpec((1, EMBED), lambda b, h: (0, 0)),
        ],
        out_specs=pl.BlockSpec((1, LQ, EMBED), lambda b, h: (b, 0, 0)),
        out_shape=jax.ShapeDtypeStruct((BATCH, LQ, EMBED), jnp.float32),
    )(x, wq3, wk3, wv3, wo3, bq2, cv)
    return (out, sorted_index)
